# baseline (speedup 1.0000x reference)
import functools

import jax
import jax.numpy as jnp
from jax import lax
from jax.experimental import pallas as pl
from jax.experimental.pallas import tpu as pltpu
from jax.experimental.pallas import tpu_sc as plsc

N = 10000
E = 320000
D_IN = 128
HIDDEN = 16
NUM_CLASSES = 40

NC = 2
NS = 16
NW = NC * NS
CHUNK = 125
CH = 80
EPT = CH * CHUNK
N_PAD = 10240
ROWS_PER_TILE = N_PAD // NS
NF = N * HIDDEN // 128
NPF = N_PAD * HIDDEN // 128
NT = N_PAD // 8
NF = N * HIDDEN // 128
NPF = N_PAD * HIDDEN // 128

_mesh = plsc.VectorSubcoreMesh(core_axis_name="c", subcore_axis_name="s")
_sc_params = pltpu.CompilerParams(use_tc_tiling_on_sc=False)
_sc_params_reg = pltpu.CompilerParams(use_tc_tiling_on_sc=False,
                                      needs_layout_passes=False)


def _zero_vmem(buf, d):
    def body(r, _):
        for l in range(d // 16):
            buf[r, pl.ds(16 * l, 16)] = jnp.zeros((16,), jnp.float32)
        return 0
    lax.fori_loop(0, 128, body, 0)


def _make_degree_kernel():
    @functools.partial(
        pl.kernel,
        out_type=jax.ShapeDtypeStruct((NW, N_PAD), jnp.float32),
        mesh=_mesh,
        compiler_params=_sc_params_reg,
        scratch_types=[
            pltpu.VMEM((EPT // 16, 16), jnp.int32),
            pltpu.VMEM((N_PAD,), jnp.float32),
        ],
    )
    def deg_kernel(col_hbm, out_hbm, col_v, hist):
        c = lax.axis_index("c")
        s = lax.axis_index("s")
        w = c * NS + s
        pltpu.sync_copy(col_hbm.at[w], col_v)

        def zero(i, _):
            hist[pl.ds(i * 16, 16)] = jnp.zeros((16,), jnp.float32)
            return 0
        lax.fori_loop(0, N_PAD // 16, zero, 0)

        ones = jnp.ones((16,), jnp.float32)

        def step(j, _):
            n = col_v[j]
            idx = (n & 7) * NT + (n >> 3)
            plsc.addupdate_scatter(hist, [idx], ones)
            return 0
        lax.fori_loop(0, EPT // 16, step, 0)
        pltpu.sync_copy(hist, out_hbm.at[w])

    return deg_kernel


def _make_agg_kernel(d):
    @functools.partial(
        pl.kernel,
        out_type=jax.ShapeDtypeStruct((NC, N_PAD, d), jnp.float32),
        mesh=_mesh,
        compiler_params=_sc_params,
        scratch_types=[
            pltpu.VMEM((CH, CHUNK), jnp.int32),
            pltpu.VMEM((CH, CHUNK), jnp.int32),
            [pltpu.VMEM((CHUNK, d), jnp.float32) for _ in range(4)],
            pltpu.VMEM((128, d), jnp.float32),
            pltpu.VMEM_SHARED((N_PAD, d), jnp.float32),
            [pltpu.SemaphoreType.DMA for _ in range(4)],
            [pltpu.SemaphoreType.DMA for _ in range(4)],
        ],
    )
    def agg_kernel(table_hbm, row_hbm, col_hbm, out_hbm,
                   row_v, col_v, bufs, zero_v, acc, gsems, ssems):
        c = lax.axis_index("c")
        s = lax.axis_index("s")
        w = c * NS + s
        pltpu.sync_copy(row_hbm.at[w], row_v)
        pltpu.sync_copy(col_hbm.at[w], col_v)
        _zero_vmem(zero_v, d)
        for i in range(ROWS_PER_TILE // 128):
            pltpu.sync_copy(zero_v, acc.at[pl.ds(s * ROWS_PER_TILE + i * 128, 128)])
        plsc.subcore_barrier()

        def gather(k, b):
            pltpu.async_copy(table_hbm.at[row_v.at[k]], bufs[b], gsems[b])

        def scatter(k, b):
            pltpu.async_copy(bufs[b], acc.at[col_v.at[k]], ssems[b], add=True)

        gather(0, 0)
        gather(1, 1)

        def step(k, b):
            pltpu.make_async_copy(table_hbm.at[row_v.at[k]], bufs[b], gsems[b]).wait()
            scatter(k, b)
            br = (b + 2) % 4

            @pl.when(k + 2 < CH)
            def _():
                @pl.when(k >= 2)
                def _():
                    pltpu.make_async_copy(
                        bufs[br], acc.at[col_v.at[k - 2]], ssems[br]).wait()
                gather(k + 2, br)

        def step4(k0, _):
            for b in range(4):
                step(k0 * 4 + b, b)
            return 0

        lax.fori_loop(0, CH // 4, step4, 0)
        for i in range(4):
            k = CH - 4 + i
            pltpu.make_async_copy(bufs[k % 4], acc.at[col_v.at[k]], ssems[k % 4]).wait()
        plsc.subcore_barrier()
        sl = pl.ds(s * ROWS_PER_TILE, ROWS_PER_TILE)
        pltpu.sync_copy(acc.at[sl], out_hbm.at[c, sl])

    return agg_kernel


_deg_kernel = _make_degree_kernel()
_agg16 = _make_agg_kernel(HIDDEN)


def _tcmm_body(x8_ref, w1b_ref, hf_ref):
    hf_ref[...] = jnp.dot(x8_ref[...], w1b_ref[...],
                          preferred_element_type=jnp.float32)[:NF]


def _tcmm(x8, W1B):
    return pl.pallas_call(
        _tcmm_body,
        out_shape=jax.ShapeDtypeStruct((NF, 128), jnp.float32),
    )(x8, W1B)


def _tc1_body(hf_ref, degt_ref, g1f_ref, dinvf_ref):
    deg8t = jnp.sum(degt_ref[...], axis=0) + 1.0
    dinv8t = lax.rsqrt(deg8t)
    jj = lax.broadcasted_iota(jnp.int32, (8, 128), 0)
    ll = lax.broadcasted_iota(jnp.int32, (8, 128), 1)
    spread = jnp.where(ll // HIDDEN == jj, 1.0, 0.0)
    dinvf = lax.dot_general(dinv8t, spread, (((0,), (0,)), ((), ())),
                            preferred_element_type=jnp.float32)[:NF]
    g1f_ref[...] = hf_ref[...] * dinvf
    dinvf_ref[...] = dinvf


def _tc1(hf, degt):
    return pl.pallas_call(
        _tc1_body,
        out_shape=[
            jax.ShapeDtypeStruct((NF, 128), jnp.float32),
            jax.ShapeDtypeStruct((NF, 128), jnp.float32),
        ],
    )(hf, degt)


def _tc2_body(s1pf_ref, g1f_ref, dinvf_ref, b1t_ref, uf_ref):
    agg = s1pf_ref[0, :NF] + s1pf_ref[1, :NF] + g1f_ref[...]
    dinvf = dinvf_ref[...]
    h = jnp.maximum(agg * dinvf + b1t_ref[...], 0.0)
    uf_ref[...] = h * dinvf


def _tc2(s1pf, g1f, dinvf, b1t):
    return pl.pallas_call(
        _tc2_body,
        out_shape=jax.ShapeDtypeStruct((NF, 128), jnp.float32),
    )(s1pf, g1f, dinvf, b1t)


def _tc3_body(s2pf_ref, uf_ref, dinvf_ref, w2_ref, b2_ref, out_ref):
    aggf = (s2pf_ref[0, :NF] + s2pf_ref[1, :NF] + uf_ref[...]) * dinvf_ref[...]
    cols = []
    for j in range(8):
        agg = aggf[:, HIDDEN * j:HIDDEN * (j + 1)]
        t = jnp.dot(agg, w2_ref[...], preferred_element_type=jnp.float32)
        t = t + b2_ref[...]
        m = jnp.max(t, axis=1, keepdims=True)
        lse = jnp.log(jnp.sum(jnp.exp(t - m), axis=1, keepdims=True))
        cols.append(t - m - lse)
    out_ref[...] = jnp.concatenate(cols, axis=1)


def _tc3(s2pf, uf, dinvf, W2, b2):
    return pl.pallas_call(
        _tc3_body,
        out_shape=jax.ShapeDtypeStruct((NF, 8 * NUM_CLASSES), jnp.float32),
    )(s2pf, uf, dinvf, W2, b2)


def kernel(x, edge_index, W1, b1, W2, b2):
    row_r = edge_index[0].reshape(NW, CH, CHUNK)
    col_r = edge_index[1].reshape(NW, CH, CHUNK)
    col_d = edge_index[1].reshape(NW, EPT // 16, 16)

    x8 = jnp.pad(x, ((0, 48), (0, 0))).reshape(1256, 8 * D_IN)
    W1B = (jnp.eye(8, dtype=jnp.float32)[:, None, :, None]
           * W1[None, :, None, :]).reshape(8 * D_IN, 128)
    b1t = jnp.tile(b1, 128 // HIDDEN)

    degp = _deg_kernel(col_d)
    hf = _tcmm(x8, W1B)
    g1f, dinvf = _tc1(hf, degp.reshape(NW, 8, NT))
    s1p = _agg16(g1f.reshape(N, HIDDEN), row_r, col_r)
    uf = _tc2(s1p.reshape(NC, NPF, 128), g1f, dinvf, b1t)
    s2p = _agg16(uf.reshape(N, HIDDEN), row_r, col_r)
    out = _tc3(s2p.reshape(NC, NPF, 128), uf, dinvf, W2, b2)
    return out.reshape(N, NUM_CLASSES)


# speedup vs baseline: 1.0164x; 1.0164x over previous
import functools

import jax
import jax.numpy as jnp
from jax import lax
from jax.experimental import pallas as pl
from jax.experimental.pallas import tpu as pltpu
from jax.experimental.pallas import tpu_sc as plsc

N = 10000
E = 320000
D_IN = 128
HIDDEN = 16
NUM_CLASSES = 40

NC = 2
NS = 16
NW = NC * NS
CHUNK = 125
CH = 80
EPT = CH * CHUNK
N_PAD = 10240
ROWS_PER_TILE = N_PAD // NS
NF = N * HIDDEN // 128
NPF = N_PAD * HIDDEN // 128
NT = N_PAD // 8

_mesh = plsc.VectorSubcoreMesh(core_axis_name="c", subcore_axis_name="s")
_sc_params = pltpu.CompilerParams(use_tc_tiling_on_sc=False)
_sc_params_reg = pltpu.CompilerParams(use_tc_tiling_on_sc=False,
                                      needs_layout_passes=False)


def _zero_vmem(buf, d):
    def body(r, _):
        for l in range(d // 16):
            buf[r, pl.ds(16 * l, 16)] = jnp.zeros((16,), jnp.float32)
        return 0
    lax.fori_loop(0, 128, body, 0)


def _make_degree_kernel():
    @functools.partial(
        pl.kernel,
        out_type=jax.ShapeDtypeStruct((NW, 8, NT), jnp.float32),
        mesh=_mesh,
        compiler_params=_sc_params_reg,
        scratch_types=[
            pltpu.VMEM((EPT // 16, 16), jnp.int32),
            pltpu.VMEM((8, NT), jnp.float32),
        ],
    )
    def deg_kernel(col_hbm, out_hbm, col_v, hist):
        c = lax.axis_index("c")
        s = lax.axis_index("s")
        w = c * NS + s
        pltpu.sync_copy(col_hbm.at[w], col_v)

        def zero(i, _):
            for j in range(8):
                hist[j, pl.ds(i * 16, 16)] = jnp.zeros((16,), jnp.float32)
            return 0
        lax.fori_loop(0, NT // 16, zero, 0)

        ones = jnp.ones((16,), jnp.float32)

        def step(j, _):
            n = col_v[j]
            plsc.addupdate_scatter(hist, [n & 7, n >> 3], ones)
            return 0
        lax.fori_loop(0, EPT // 16, step, 0)
        pltpu.sync_copy(hist, out_hbm.at[w])

    return deg_kernel


def _make_agg_kernel(d):
    @functools.partial(
        pl.kernel,
        out_type=jax.ShapeDtypeStruct((NC, N_PAD, d), jnp.float32),
        mesh=_mesh,
        compiler_params=_sc_params,
        scratch_types=[
            pltpu.VMEM((CH, CHUNK), jnp.int32),
            pltpu.VMEM((CH, CHUNK), jnp.int32),
            [pltpu.VMEM((CHUNK, d), jnp.float32) for _ in range(4)],
            pltpu.VMEM((128, d), jnp.float32),
            pltpu.VMEM_SHARED((N_PAD, d), jnp.float32),
            [pltpu.SemaphoreType.DMA for _ in range(4)],
            [pltpu.SemaphoreType.DMA for _ in range(4)],
        ],
    )
    def agg_kernel(table_hbm, row_hbm, col_hbm, out_hbm,
                   row_v, col_v, bufs, zero_v, acc, gsems, ssems):
        c = lax.axis_index("c")
        s = lax.axis_index("s")
        w = c * NS + s
        pltpu.sync_copy(row_hbm.at[w], row_v)
        pltpu.sync_copy(col_hbm.at[w], col_v)
        _zero_vmem(zero_v, d)
        for i in range(ROWS_PER_TILE // 128):
            pltpu.sync_copy(zero_v, acc.at[pl.ds(s * ROWS_PER_TILE + i * 128, 128)])
        plsc.subcore_barrier()

        def gather(k, b):
            pltpu.async_copy(table_hbm.at[row_v.at[k]], bufs[b], gsems[b])

        def scatter(k, b):
            pltpu.async_copy(bufs[b], acc.at[col_v.at[k]], ssems[b], add=True)

        gather(0, 0)
        gather(1, 1)

        def step(k, b):
            pltpu.make_async_copy(table_hbm.at[row_v.at[k]], bufs[b], gsems[b]).wait()
            scatter(k, b)
            br = (b + 2) % 4

            @pl.when(k + 2 < CH)
            def _():
                @pl.when(k >= 2)
                def _():
                    pltpu.make_async_copy(
                        bufs[br], acc.at[col_v.at[k - 2]], ssems[br]).wait()
                gather(k + 2, br)

        def step4(k0, _):
            for b in range(4):
                step(k0 * 4 + b, b)
            return 0

        lax.fori_loop(0, CH // 4, step4, 0)
        for i in range(4):
            k = CH - 4 + i
            pltpu.make_async_copy(bufs[k % 4], acc.at[col_v.at[k]], ssems[k % 4]).wait()
        plsc.subcore_barrier()
        sl = pl.ds(s * ROWS_PER_TILE, ROWS_PER_TILE)
        pltpu.sync_copy(acc.at[sl], out_hbm.at[c, sl])

    return agg_kernel


_deg_kernel = _make_degree_kernel()
_agg16 = _make_agg_kernel(HIDDEN)


def _tc1_body(x8_ref, w1b_ref, degt_ref, g1f_ref, dinvf_ref):
    deg8t = jnp.sum(degt_ref[...], axis=0) + 1.0
    dinv8t = lax.rsqrt(deg8t)
    jj = lax.broadcasted_iota(jnp.int32, (8, 128), 0)
    ll = lax.broadcasted_iota(jnp.int32, (8, 128), 1)
    spread = jnp.where(ll // HIDDEN == jj, 1.0, 0.0)
    dinvf = lax.dot_general(dinv8t, spread, (((0,), (0,)), ((), ())),
                            preferred_element_type=jnp.float32)[:NF]
    hf = jnp.dot(x8_ref[...], w1b_ref[...],
                 preferred_element_type=jnp.float32)[:NF]
    g1f_ref[...] = hf * dinvf
    dinvf_ref[...] = dinvf


def _tc1(x8, W1B, degt):
    return pl.pallas_call(
        _tc1_body,
        out_shape=[
            jax.ShapeDtypeStruct((NF, 128), jnp.float32),
            jax.ShapeDtypeStruct((NF, 128), jnp.float32),
        ],
    )(x8, W1B, degt)


def _tc2_body(s1pf_ref, g1f_ref, dinvf_ref, b1t_ref, uf_ref):
    agg = s1pf_ref[0, :NF] + s1pf_ref[1, :NF] + g1f_ref[...]
    dinvf = dinvf_ref[...]
    h = jnp.maximum(agg * dinvf + b1t_ref[...], 0.0)
    uf_ref[...] = h * dinvf


def _tc2(s1pf, g1f, dinvf, b1t):
    return pl.pallas_call(
        _tc2_body,
        out_shape=jax.ShapeDtypeStruct((NF, 128), jnp.float32),
    )(s1pf, g1f, dinvf, b1t)


def _tc3_body(s2pf_ref, uf_ref, dinvf_ref, w2_ref, b2_ref, out_ref):
    aggf = (s2pf_ref[0, :NF] + s2pf_ref[1, :NF] + uf_ref[...]) * dinvf_ref[...]
    cols = []
    for j in range(8):
        agg = aggf[:, HIDDEN * j:HIDDEN * (j + 1)]
        t = jnp.dot(agg, w2_ref[...], preferred_element_type=jnp.float32)
        t = t + b2_ref[...]
        m = jnp.max(t, axis=1, keepdims=True)
        lse = jnp.log(jnp.sum(jnp.exp(t - m), axis=1, keepdims=True))
        cols.append(t - m - lse)
    out_ref[...] = jnp.concatenate(cols, axis=1)


def _tc3(s2pf, uf, dinvf, W2, b2):
    return pl.pallas_call(
        _tc3_body,
        out_shape=jax.ShapeDtypeStruct((NF, 8 * NUM_CLASSES), jnp.float32),
    )(s2pf, uf, dinvf, W2, b2)


def kernel(x, edge_index, W1, b1, W2, b2):
    row_r = edge_index[0].reshape(NW, CH, CHUNK)
    col_r = edge_index[1].reshape(NW, CH, CHUNK)
    col_d = edge_index[1].reshape(NW, EPT // 16, 16)

    x8 = jnp.pad(x, ((0, 48), (0, 0))).reshape(1256, 8 * D_IN)
    W1B = (jnp.eye(8, dtype=jnp.float32)[:, None, :, None]
           * W1[None, :, None, :]).reshape(8 * D_IN, 128)
    b1t = jnp.tile(b1, 128 // HIDDEN)

    degp = _deg_kernel(col_d)
    g1f, dinvf = _tc1(x8, W1B, degp)
    s1p = _agg16(g1f.reshape(N, HIDDEN), row_r, col_r)
    uf = _tc2(s1p.reshape(NC, NPF, 128), g1f, dinvf, b1t)
    s2p = _agg16(uf.reshape(N, HIDDEN), row_r, col_r)
    out = _tc3(s2p.reshape(NC, NPF, 128), uf, dinvf, W2, b2)
    return out.reshape(N, NUM_CLASSES)


# speedup vs baseline: 1.0414x; 1.0246x over previous
import functools

import jax
import jax.numpy as jnp
from jax import lax
from jax.experimental import pallas as pl
from jax.experimental.pallas import tpu as pltpu
from jax.experimental.pallas import tpu_sc as plsc

N = 10000
E = 320000
D_IN = 128
HIDDEN = 16
NUM_CLASSES = 40

NC = 2
NS = 16
NW = NC * NS
CHUNK = 125
CH = 80
EPT = CH * CHUNK
N_PAD = 10240
ROWS_PER_TILE = N_PAD // NS
NF = N * HIDDEN // 128
NPF = N_PAD * HIDDEN // 128
NT = N_PAD // 8

_mesh = plsc.VectorSubcoreMesh(core_axis_name="c", subcore_axis_name="s")
_sc_params = pltpu.CompilerParams(use_tc_tiling_on_sc=False)
_sc_params_reg = pltpu.CompilerParams(use_tc_tiling_on_sc=False,
                                      needs_layout_passes=False)


def _zero_vmem(buf, d):
    def body(r, _):
        for l in range(d // 16):
            buf[r, pl.ds(16 * l, 16)] = jnp.zeros((16,), jnp.float32)
        return 0
    lax.fori_loop(0, 128, body, 0)


def _make_degree_kernel():
    @functools.partial(
        pl.kernel,
        out_type=jax.ShapeDtypeStruct((NW, 8, NT), jnp.float32),
        mesh=_mesh,
        compiler_params=_sc_params_reg,
        scratch_types=[
            pltpu.VMEM((EPT // 16, 16), jnp.int32),
            pltpu.VMEM((8, NT), jnp.float32),
        ],
    )
    def deg_kernel(col_hbm, out_hbm, col_v, hist):
        c = lax.axis_index("c")
        s = lax.axis_index("s")
        w = c * NS + s
        pltpu.sync_copy(col_hbm.at[w], col_v)

        def zero(i, _):
            for j in range(8):
                hist[j, pl.ds(i * 16, 16)] = jnp.zeros((16,), jnp.float32)
            return 0
        lax.fori_loop(0, NT // 16, zero, 0)

        ones = jnp.ones((16,), jnp.float32)

        def step(j, _):
            n = col_v[j]
            plsc.addupdate_scatter(hist, [n & 7, n >> 3], ones)
            return 0
        lax.fori_loop(0, EPT // 16, step, 0)
        pltpu.sync_copy(hist, out_hbm.at[w])

    return deg_kernel


def _make_agg_kernel(d):
    @functools.partial(
        pl.kernel,
        out_type=jax.ShapeDtypeStruct((NC, N_PAD, d), jnp.float32),
        mesh=_mesh,
        compiler_params=_sc_params,
        scratch_types=[
            pltpu.VMEM((CH, CHUNK), jnp.int32),
            pltpu.VMEM((CH, CHUNK), jnp.int32),
            [pltpu.VMEM((CHUNK, d), jnp.float32) for _ in range(4)],
            pltpu.VMEM((128, d), jnp.float32),
            pltpu.VMEM_SHARED((N_PAD, d), jnp.float32),
            [pltpu.SemaphoreType.DMA for _ in range(4)],
            [pltpu.SemaphoreType.DMA for _ in range(4)],
        ],
    )
    def agg_kernel(table_hbm, row_hbm, col_hbm, out_hbm,
                   row_v, col_v, bufs, zero_v, acc, gsems, ssems):
        c = lax.axis_index("c")
        s = lax.axis_index("s")
        w = c * NS + s
        pltpu.sync_copy(row_hbm.at[w], row_v)
        pltpu.sync_copy(col_hbm.at[w], col_v)
        _zero_vmem(zero_v, d)
        for i in range(ROWS_PER_TILE // 128):
            pltpu.sync_copy(zero_v, acc.at[pl.ds(s * ROWS_PER_TILE + i * 128, 128)])
        plsc.subcore_barrier()

        def gather(k, b):
            pltpu.async_copy(table_hbm.at[row_v.at[k]], bufs[b], gsems[b])

        def scatter(k, b):
            pltpu.async_copy(bufs[b], acc.at[col_v.at[k]], ssems[b], add=True)

        gather(0, 0)
        gather(1, 1)

        def step(k, b):
            pltpu.make_async_copy(table_hbm.at[row_v.at[k]], bufs[b], gsems[b]).wait()
            scatter(k, b)
            br = (b + 2) % 4

            @pl.when(k + 2 < CH)
            def _():
                @pl.when(k >= 2)
                def _():
                    pltpu.make_async_copy(
                        bufs[br], acc.at[col_v.at[k - 2]], ssems[br]).wait()
                gather(k + 2, br)

        def step4(k0, _):
            for b in range(4):
                step(k0 * 4 + b, b)
            return 0

        lax.fori_loop(0, CH // 4, step4, 0)
        for i in range(4):
            k = CH - 4 + i
            pltpu.make_async_copy(bufs[k % 4], acc.at[col_v.at[k]], ssems[k % 4]).wait()
        plsc.subcore_barrier()
        sl = pl.ds(s * ROWS_PER_TILE, ROWS_PER_TILE)
        pltpu.sync_copy(acc.at[sl], out_hbm.at[c, sl])

    return agg_kernel


_deg_kernel = _make_degree_kernel()
_agg16 = _make_agg_kernel(HIDDEN)


def _tc1_body(x8_ref, w1b_ref, degt_ref, g1f_ref, dinvf_ref):
    deg8t = jnp.sum(degt_ref[...], axis=0) + 1.0
    dinv8t = lax.rsqrt(deg8t)
    jj = lax.broadcasted_iota(jnp.int32, (8, 128), 0)
    ll = lax.broadcasted_iota(jnp.int32, (8, 128), 1)
    spread = jnp.where(ll // HIDDEN == jj, 1.0, 0.0)
    dinvf = lax.dot_general(dinv8t, spread, (((0,), (0,)), ((), ())),
                            preferred_element_type=jnp.float32)[:NF]
    hf = jnp.dot(x8_ref[...], w1b_ref[...],
                 preferred_element_type=jnp.float32)[:NF]
    g1f_ref[...] = hf * dinvf
    dinvf_ref[...] = dinvf


def _tc1(x8, W1B, degt):
    return pl.pallas_call(
        _tc1_body,
        out_shape=[
            jax.ShapeDtypeStruct((NF, 128), jnp.float32),
            jax.ShapeDtypeStruct((NF, 128), jnp.float32),
        ],
    )(x8, W1B, degt)


def _tc2_body(s1pf_ref, g1f_ref, dinvf_ref, b1t_ref, uf_ref):
    agg = s1pf_ref[0, :NF] + s1pf_ref[1, :NF] + g1f_ref[...]
    dinvf = dinvf_ref[...]
    h = jnp.maximum(agg * dinvf + b1t_ref[...], 0.0)
    uf_ref[...] = h * dinvf


def _tc2(s1pf, g1f, dinvf, b1t):
    return pl.pallas_call(
        _tc2_body,
        out_shape=jax.ShapeDtypeStruct((NF, 128), jnp.float32),
    )(s1pf, g1f, dinvf, b1t)


def _tc3_body(s2pf_ref, uf_ref, dinvf_ref, w2b_ref, b2t_ref, out_ref):
    nc = NUM_CLASSES
    aggf = (s2pf_ref[0, :NF] + s2pf_ref[1, :NF] + uf_ref[...]) * dinvf_ref[...]
    t = jnp.dot(aggf, w2b_ref[...], preferred_element_type=jnp.float32)
    t = t + b2t_ref[...]
    m8 = jnp.concatenate(
        [jnp.max(t[:, nc * j:nc * (j + 1)], axis=1, keepdims=True)
         for j in range(8)], axis=1)
    jj = lax.broadcasted_iota(jnp.int32, (8, 8 * nc), 0)
    ll = lax.broadcasted_iota(jnp.int32, (8, 8 * nc), 1)
    spread = jnp.where(ll // nc == jj, 1.0, 0.0)
    ms = jnp.dot(m8, spread, preferred_element_type=jnp.float32)
    em = jnp.exp(t - ms)
    sums = lax.dot_general(em, spread, (((1,), (1,)), ((), ())),
                           preferred_element_type=jnp.float32)
    lse = jnp.dot(jnp.log(sums), spread, preferred_element_type=jnp.float32)
    out_ref[...] = t - ms - lse


def _tc3(s2pf, uf, dinvf, W2B2, b2t):
    return pl.pallas_call(
        _tc3_body,
        out_shape=jax.ShapeDtypeStruct((NF, 8 * NUM_CLASSES), jnp.float32),
    )(s2pf, uf, dinvf, W2B2, b2t)


def kernel(x, edge_index, W1, b1, W2, b2):
    row_r = edge_index[0].reshape(NW, CH, CHUNK)
    col_r = edge_index[1].reshape(NW, CH, CHUNK)
    col_d = edge_index[1].reshape(NW, EPT // 16, 16)

    x8 = jnp.pad(x, ((0, 48), (0, 0))).reshape(1256, 8 * D_IN)
    W1B = (jnp.eye(8, dtype=jnp.float32)[:, None, :, None]
           * W1[None, :, None, :]).reshape(8 * D_IN, 128)
    b1t = jnp.tile(b1, 128 // HIDDEN)

    degp = _deg_kernel(col_d)
    g1f, dinvf = _tc1(x8, W1B, degp)
    s1p = _agg16(g1f.reshape(N, HIDDEN), row_r, col_r)
    uf = _tc2(s1p.reshape(NC, NPF, 128), g1f, dinvf, b1t)
    W2B2 = (jnp.eye(8, dtype=jnp.float32)[:, None, :, None]
            * W2[None, :, None, :]).reshape(128, 8 * NUM_CLASSES)
    b2t = jnp.tile(b2, 8)
    s2p = _agg16(uf.reshape(N, HIDDEN), row_r, col_r)
    out = _tc3(s2p.reshape(NC, NPF, 128), uf, dinvf, W2B2, b2t)
    return out.reshape(N, NUM_CLASSES)
